# baseline (speedup 1.0000x reference)
import functools

import jax
import jax.numpy as jnp
from jax import lax
from jax.experimental import pallas as pl
from jax.experimental.pallas import tpu as pltpu
from jax.experimental.pallas import tpu_sc as plsc

_B, _T = 1024, 200
_N = _B * _T
_DIMS = (16, 16, 8, 32, 8, 16, 8, 16, 8)
_NF = len(_DIMS)

_NC, _NS = 2, 16
_NW = _NC * _NS
_NTOK = _N // _NW
_CH = 128
_G = 5
_STEP = _CH * _G
_NSTEP = _NTOK // _STEP


def _sc_body(xt_ref, *rest):
    w_refs = rest[:_NF]
    out_refs = rest[_NF:2 * _NF]
    idx_ref = rest[2 * _NF]
    row_refs = rest[2 * _NF + 1:3 * _NF + 1]
    gsem, wsem = rest[3 * _NF + 1], rest[3 * _NF + 2]

    wid = lax.axis_index("s") * _NC + lax.axis_index("c")
    base = wid * _NTOK

    @pl.loop(0, _NSTEP)
    def _step(s):
        t0 = base + s * _STEP
        pltpu.sync_copy(xt_ref.at[pl.ds(t0 // _STEP, 1), :, :], idx_ref)
        handles = []
        for i in range(_NF):
            handles.append(pltpu.async_copy(
                w_refs[i].at[idx_ref.at[0, i]],
                row_refs[i],
                gsem))
        for h in handles:
            h.wait()
        wh = [pltpu.async_copy(row_refs[i],
                               out_refs[i].at[pl.ds(t0, _STEP), :], wsem)
              for i in range(_NF)]
        for h in wh:
            h.wait()


@jax.jit
def kernel(x, W_msg, W_act, W_finish, W_effect, W_phase, W_position,
           W_number, W_place, W_attrib):
    Ws = (W_msg, W_act, W_finish, W_effect, W_phase, W_position,
          W_number, W_place, W_attrib)
    xt = x.reshape(_N // _STEP, _STEP, _NF).transpose(0, 2, 1)

    mesh = plsc.VectorSubcoreMesh(core_axis_name="c", subcore_axis_name="s",
                                  num_cores=_NC, num_subcores=_NS)
    out_type = [jax.ShapeDtypeStruct((_N, d), jnp.float32) for d in _DIMS]
    scratch = ([pltpu.VMEM((1, _NF, _STEP), jnp.int32)]
               + [pltpu.VMEM((_STEP, d), jnp.float32) for d in _DIMS]
               + [pltpu.SemaphoreType.DMA, pltpu.SemaphoreType.DMA])
    outs = pl.kernel(
        _sc_body,
        out_type=out_type,
        mesh=mesh,
        scratch_types=scratch,
        compiler_params=pltpu.CompilerParams(use_tc_tiling_on_sc=False),
    )(xt, *Ws)
    return tuple(o.reshape(_B, _T, d) for o, d in zip(outs, _DIMS))


# speedup vs baseline: 2.4472x; 2.4472x over previous
import jax
import jax.numpy as jnp
from jax import lax
from jax.experimental import pallas as pl
from jax.experimental.pallas import tpu as pltpu
from jax.experimental.pallas import tpu_sc as plsc

_B, _T = 1024, 200
_N = _B * _T
_DIMS = (16, 16, 8, 32, 8, 16, 8, 16, 8)
_NF = len(_DIMS)

_NC, _NS = 2, 16
_NW = _NC * _NS
_NTOK = _N // _NW
_STEP = 128
_NSTEP = _NTOK // _STEP
_NG = _STEP // 16


def _sc_body(xt_ref, *rest):
    w_hbm = rest[:_NF]
    out_refs = rest[_NF:2 * _NF]
    idx_ref = rest[2 * _NF]
    wv = rest[2 * _NF + 1:3 * _NF + 1]
    rows = (rest[3 * _NF + 1:4 * _NF + 1], rest[4 * _NF + 1:5 * _NF + 1])
    wsem = rest[5 * _NF + 1:5 * _NF + 3]

    wid = lax.axis_index("s") * _NC + lax.axis_index("c")
    base = wid * _NTOK

    for i in range(_NF):
        pltpu.sync_copy(w_hbm[i], wv[i])
    pltpu.sync_copy(xt_ref.at[pl.ds(wid * _NSTEP, _NSTEP), :, :], idx_ref)

    iota = lax.iota(jnp.int32, 16)
    cols = [jnp.full((16,), d, jnp.int32) for d in range(max(_DIMS))]

    def compute(step, b):
        @pl.loop(0, _NG)
        def _grp(g):
            tok16 = iota + g * 16
            for i in range(_NF):
                idx16 = idx_ref[step, i, pl.ds(g * 16, 16)]
                for d in range(_DIMS[i]):
                    vals = plsc.load_gather(wv[i], [idx16, cols[d]])
                    plsc.store_scatter(rows[b][i], [tok16, cols[d]], vals)
        t0 = base + step * _STEP
        for i in range(_NF):
            pltpu.async_copy(rows[b][i], out_refs[i].at[pl.ds(t0, _STEP), :],
                             wsem[b])

    def drain(b):
        for i in range(_NF):
            pltpu.make_async_copy(
                rows[b][i], out_refs[i].at[pl.ds(base, _STEP), :],
                wsem[b]).wait()

    compute(0, 0)
    compute(1, 1)

    @pl.loop(2, _NSTEP, step=2)
    def _steps(s):
        for b in range(2):
            drain(b)
            compute(s + b, b)

    for b in range(2):
        drain(b)


@jax.jit
def kernel(x, W_msg, W_act, W_finish, W_effect, W_phase, W_position,
           W_number, W_place, W_attrib):
    Ws = (W_msg, W_act, W_finish, W_effect, W_phase, W_position,
          W_number, W_place, W_attrib)
    xt = x.reshape(_N // _STEP, _STEP, _NF).transpose(0, 2, 1)

    mesh = plsc.VectorSubcoreMesh(core_axis_name="c", subcore_axis_name="s",
                                  num_cores=_NC, num_subcores=_NS)
    out_type = [jax.ShapeDtypeStruct((_N, d), jnp.float32) for d in _DIMS]
    scratch = ([pltpu.VMEM((_NSTEP, _NF, _STEP), jnp.int32)]
               + [pltpu.VMEM((v, d), jnp.float32)
                  for v, d in zip((30, 10, 3, 256, 4, 9, 13, 31, 10), _DIMS)]
               + [pltpu.VMEM((_STEP, d), jnp.float32) for d in _DIMS]
               + [pltpu.VMEM((_STEP, d), jnp.float32) for d in _DIMS]
               + [pltpu.SemaphoreType.DMA, pltpu.SemaphoreType.DMA])
    outs = pl.kernel(
        _sc_body,
        out_type=out_type,
        mesh=mesh,
        scratch_types=scratch,
        compiler_params=pltpu.CompilerParams(use_tc_tiling_on_sc=False,
                                             needs_layout_passes=False),
    )(xt, *Ws)
    return tuple(o.reshape(_B, _T, d) for o, d in zip(outs, _DIMS))
